# baseline (device time: 35472 ns/iter reference)
import jax
import jax.numpy as jnp
from jax import lax
from jax.experimental import pallas as pl
from jax.experimental.pallas import tpu as pltpu

N_DEV = 32
B = 2
SQ = 128
SKV = 128
HQ_LOCAL = 4
DH = 64
D_MODEL = 512
HD_LOCAL = HQ_LOCAL * DH
ROWS = B * SQ

SLOT_RSX = 0
SLOT_RSY = 1
SLOT_ARZ = 4
SLOT_AGY = 7
SLOT_AGX = 10
N_SLOTS = 11


def kernel(x, Wq, K_ext, V_ext, Wo):
    my = lax.axis_index("i")
    wq_s = lax.dynamic_slice_in_dim(Wq, my * HD_LOCAL, HD_LOCAL, axis=1)
    wo_s = lax.dynamic_slice_in_dim(Wo, my * HD_LOCAL, HD_LOCAL, axis=0)
    k_l = K_ext.transpose(0, 2, 1, 3).reshape(B * HQ_LOCAL, SKV, DH)
    v_l = V_ext.transpose(0, 2, 1, 3).reshape(B * HQ_LOCAL, SKV, DH)

    def body(x_ref, wq_ref, k_ref, v_ref, wo_ref, out_ref,
             acc_ref, comm_ref, send_sems, recv_sems):
        my_pos = lax.axis_index("i")
        zz = my_pos // 8
        jj = lax.rem(my_pos, 8)
        yy = jj // 2
        xx = lax.rem(jj + yy, 2)

        def lidx(px, py, pz):
            return pz * 8 + py * 2 + lax.rem(px + py, 2)

        x_partner = lidx(1 - xx, yy, zz)
        y_partners = [lidx(xx, lax.rem(yy + d, 4), zz) for d in (1, 2, 3)]
        z_partners = [lidx(xx, yy, lax.rem(zz + d, 4)) for d in (1, 2, 3)]
        all_partners = [x_partner] + y_partners + z_partners

        barrier = pltpu.get_barrier_semaphore()
        for p in all_partners:
            pl.semaphore_signal(
                barrier, inc=1,
                device_id=(p,), device_id_type=pl.DeviceIdType.MESH,
            )
        pl.semaphore_wait(barrier, len(all_partners))

        def copy(slot, rows, dst_dev, src_off):
            return pltpu.make_async_remote_copy(
                src_ref=acc_ref.at[pl.ds(src_off, rows), :],
                dst_ref=comm_ref.at[slot, pl.ds(0, rows), :],
                send_sem=send_sems.at[slot],
                recv_sem=recv_sems.at[slot],
                device_id=(dst_dev,),
                device_id_type=pl.DeviceIdType.MESH,
            )

        xs = x_ref[...].reshape(ROWS, D_MODEL)
        q = jnp.dot(xs, wq_ref[...], preferred_element_type=jnp.float32)

        rowb = lax.broadcasted_iota(jnp.int32, (SQ, SKV), 0) // 64
        colb = lax.broadcasted_iota(jnp.int32, (SQ, SKV), 1) // 64
        mask = rowb == colb

        rsx = copy(SLOT_RSX, 128, x_partner, (1 - xx) * 128)

        def partial_b(b):
            ctx_h = []
            for h in range(HQ_LOCAL):
                qbh = q[b * SQ:(b + 1) * SQ, h * DH:(h + 1) * DH]
                kbh = k_ref[b * HQ_LOCAL + h]
                vbh = v_ref[b * HQ_LOCAL + h]
                s = lax.dot_general(
                    qbh, kbh, (((1,), (1,)), ((), ())),
                    preferred_element_type=jnp.float32,
                ) * 0.125
                s = jnp.where(mask, s, -1e9)
                m = jnp.max(s, axis=1, keepdims=True)
                w = jnp.exp(s - m)
                w = w / jnp.sum(w, axis=1, keepdims=True)
                ctx_h.append(jnp.dot(w, vbh, preferred_element_type=jnp.float32))
            ctx = jnp.concatenate(ctx_h, axis=1)
            return jnp.dot(ctx, wo_ref[...],
                           preferred_element_type=jnp.float32)

        acc_ref[0:128, :] = partial_b(0)

        @pl.when(xx == 1)
        def _():
            rsx.start()

        acc_ref[128:256, :] = partial_b(1)

        @pl.when(xx == 0)
        def _():
            rsx.start()

        keep_x = xx * 128
        rsx.wait()
        acc_ref[pl.ds(keep_x, 128), :] = (
            acc_ref[pl.ds(keep_x, 128), :] + comm_ref[SLOT_RSX, :128, :]
        )

        keep_y = keep_x + yy * 32
        rs_y = []
        for d, p in zip((1, 2, 3), y_partners):
            slot = SLOT_RSY + (4 - d) - 1
            yp = lax.rem(yy + d, 4)
            r = copy(slot, 32, p, keep_x + yp * 32)
            r.start()
            rs_y.append(r)
        for r in rs_y:
            r.wait_recv()
        acc_ref[pl.ds(keep_y, 32), :] = (
            acc_ref[pl.ds(keep_y, 32), :]
            + comm_ref[SLOT_RSY + 0, :32, :]
            + comm_ref[SLOT_RSY + 1, :32, :]
            + comm_ref[SLOT_RSY + 2, :32, :]
        )
        for r in rs_y:
            r.wait_send()

        ar_z = []
        for d, p in zip((1, 2, 3), z_partners):
            slot = SLOT_ARZ + (4 - d) - 1
            r = copy(slot, 32, p, keep_y)
            r.start()
            ar_z.append(r)
        for r in ar_z:
            r.wait_recv()
        acc_ref[pl.ds(keep_y, 32), :] = (
            acc_ref[pl.ds(keep_y, 32), :]
            + comm_ref[SLOT_ARZ + 0, :32, :]
            + comm_ref[SLOT_ARZ + 1, :32, :]
            + comm_ref[SLOT_ARZ + 2, :32, :]
        )
        for r in ar_z:
            r.wait_send()

        ag_y = []
        for d, p in zip((1, 2, 3), y_partners):
            slot = SLOT_AGY + (4 - d) - 1
            r = copy(slot, 32, p, keep_y)
            r.start()
            ag_y.append(r)
        for r in ag_y:
            r.wait_recv()
        for d in (1, 2, 3):
            ys = lax.rem(yy + d, 4)
            acc_ref[pl.ds(keep_x + ys * 32, 32), :] = comm_ref[SLOT_AGY + d - 1, :32, :]
        for r in ag_y:
            r.wait_send()

        agx = copy(SLOT_AGX, 128, x_partner, keep_x)
        agx.start()

        @pl.when(xx == 0)
        def _():
            out_ref[0, :, :] = acc_ref[0:128, :]

        @pl.when(xx == 1)
        def _():
            out_ref[1, :, :] = acc_ref[128:256, :]

        agx.wait()

        @pl.when(xx == 0)
        def _():
            out_ref[1, :, :] = comm_ref[SLOT_AGX, :128, :]

        @pl.when(xx == 1)
        def _():
            out_ref[0, :, :] = comm_ref[SLOT_AGX, :128, :]

    return pl.pallas_call(
        body,
        out_shape=jax.ShapeDtypeStruct((B, SQ, D_MODEL), jnp.float32),
        in_specs=[pl.BlockSpec(memory_space=pltpu.VMEM)] * 5,
        out_specs=pl.BlockSpec(memory_space=pltpu.VMEM),
        scratch_shapes=[
            pltpu.VMEM((ROWS, D_MODEL), jnp.float32),
            pltpu.VMEM((N_SLOTS, 128, D_MODEL), jnp.float32),
            pltpu.SemaphoreType.DMA((N_SLOTS,)),
            pltpu.SemaphoreType.DMA((N_SLOTS,)),
        ],
        compiler_params=pltpu.CompilerParams(collective_id=0),
    )(x, wq_s, k_l, v_l, wo_s)


# device time: 33915 ns/iter; 1.0459x vs baseline; 1.0459x over previous
import jax
import jax.numpy as jnp
from jax import lax
from jax.experimental import pallas as pl
from jax.experimental.pallas import tpu as pltpu

N_DEV = 32
B = 2
SQ = 128
SKV = 128
HQ_LOCAL = 4
DH = 64
D_MODEL = 512
HD_LOCAL = HQ_LOCAL * DH
ROWS = B * SQ

SLOT_RSXP = 0
SLOT_RSXQ = 1
SLOT_AGXP = 2
SLOT_AGXQ = 3
MS_P_RSY = 0
MS_Q_RSZ = 3
MS_P_ARZ = 6
MS_Q_ARY = 9
MS_P_AGY = 12
MS_Q_AGZ = 15
N_MID = 18
N_SEMS = 4 + N_MID


def kernel(x, Wq, K_ext, V_ext, Wo):
    my = lax.axis_index("i")
    wq_s = lax.dynamic_slice_in_dim(Wq, my * HD_LOCAL, HD_LOCAL, axis=1)
    wo_s = lax.dynamic_slice_in_dim(Wo, my * HD_LOCAL, HD_LOCAL, axis=0)
    k_l = K_ext.transpose(0, 2, 1, 3).reshape(B * HQ_LOCAL, SKV, DH)
    v_l = V_ext.transpose(0, 2, 1, 3).reshape(B * HQ_LOCAL, SKV, DH)

    def body(x_ref, wq_ref, k_ref, v_ref, wo_ref, out_ref,
             acc_ref, commx_ref, commm_ref, send_sems, recv_sems):
        my_pos = lax.axis_index("i")
        zz = my_pos // 8
        jj = lax.rem(my_pos, 8)
        yy = jj // 2
        xx = lax.rem(jj + yy, 2)

        def lidx(px, py, pz):
            return pz * 8 + py * 2 + lax.rem(px + py, 2)

        x_partner = lidx(1 - xx, yy, zz)
        y_partners = [lidx(xx, lax.rem(yy + d, 4), zz) for d in (1, 2, 3)]
        z_partners = [lidx(xx, yy, lax.rem(zz + d, 4)) for d in (1, 2, 3)]
        all_partners = [x_partner] + y_partners + z_partners

        barrier = pltpu.get_barrier_semaphore()
        for p in all_partners:
            pl.semaphore_signal(
                barrier, inc=1,
                device_id=(p,), device_id_type=pl.DeviceIdType.MESH,
            )
        pl.semaphore_wait(barrier, len(all_partners))

        def copy_x(slot, dst_dev, src_off):
            return pltpu.make_async_remote_copy(
                src_ref=acc_ref.at[pl.ds(src_off, 64), :],
                dst_ref=commx_ref.at[slot],
                send_sem=send_sems.at[slot],
                recv_sem=recv_sems.at[slot],
                device_id=(dst_dev,),
                device_id_type=pl.DeviceIdType.MESH,
            )

        def copy_m(mslot, dst_dev, src_off):
            return pltpu.make_async_remote_copy(
                src_ref=acc_ref.at[pl.ds(src_off, 16), :],
                dst_ref=commm_ref.at[mslot],
                send_sem=send_sems.at[4 + mslot],
                recv_sem=recv_sems.at[4 + mslot],
                device_id=(dst_dev,),
                device_id_type=pl.DeviceIdType.MESH,
            )

        xs = x_ref[...].reshape(ROWS, D_MODEL)
        q = jnp.dot(xs, wq_ref[...], preferred_element_type=jnp.float32)

        rowb = lax.broadcasted_iota(jnp.int32, (SQ, SKV), 0) // 64
        colb = lax.broadcasted_iota(jnp.int32, (SQ, SKV), 1) // 64
        mask = rowb == colb

        rsxp = copy_x(SLOT_RSXP, x_partner, (1 - xx) * 128)
        rsxq = copy_x(SLOT_RSXQ, x_partner, (1 - xx) * 128 + 64)

        def partial_b(b):
            ctx_h = []
            for h in range(HQ_LOCAL):
                qbh = q[b * SQ:(b + 1) * SQ, h * DH:(h + 1) * DH]
                kbh = k_ref[b * HQ_LOCAL + h]
                vbh = v_ref[b * HQ_LOCAL + h]
                s = lax.dot_general(
                    qbh, kbh, (((1,), (1,)), ((), ())),
                    preferred_element_type=jnp.float32,
                ) * 0.125
                s = jnp.where(mask, s, -1e9)
                m = jnp.max(s, axis=1, keepdims=True)
                w = jnp.exp(s - m)
                w = w / jnp.sum(w, axis=1, keepdims=True)
                ctx_h.append(jnp.dot(w, vbh, preferred_element_type=jnp.float32))
            ctx = jnp.concatenate(ctx_h, axis=1)
            return jnp.dot(ctx, wo_ref[...],
                           preferred_element_type=jnp.float32)

        acc_ref[0:128, :] = partial_b(0)

        @pl.when(xx == 1)
        def _():
            rsxp.start()
            rsxq.start()

        acc_ref[128:256, :] = partial_b(1)

        @pl.when(xx == 0)
        def _():
            rsxp.start()
            rsxq.start()

        keep_x = xx * 128
        keep_p = keep_x + yy * 16
        keep_q = keep_x + 64 + zz * 16

        rsxp.wait_recv()
        acc_ref[pl.ds(keep_x, 64), :] = (
            acc_ref[pl.ds(keep_x, 64), :] + commx_ref[SLOT_RSXP]
        )

        s1 = []
        for d, p in zip((1, 2, 3), y_partners):
            yp = lax.rem(yy + d, 4)
            r = copy_m(MS_P_RSY + 3 - d, p, keep_x + yp * 16)
            r.start()
            s1.append(r)
        rsxq.wait_recv()
        acc_ref[pl.ds(keep_x + 64, 64), :] = (
            acc_ref[pl.ds(keep_x + 64, 64), :] + commx_ref[SLOT_RSXQ]
        )
        for d, p in zip((1, 2, 3), z_partners):
            zp = lax.rem(zz + d, 4)
            r = copy_m(MS_Q_RSZ + 3 - d, p, keep_x + 64 + zp * 16)
            r.start()
            s1.append(r)
        for r in s1:
            r.wait_recv()
        acc_ref[pl.ds(keep_p, 16), :] = (
            acc_ref[pl.ds(keep_p, 16), :]
            + commm_ref[MS_P_RSY + 0]
            + commm_ref[MS_P_RSY + 1]
            + commm_ref[MS_P_RSY + 2]
        )
        acc_ref[pl.ds(keep_q, 16), :] = (
            acc_ref[pl.ds(keep_q, 16), :]
            + commm_ref[MS_Q_RSZ + 0]
            + commm_ref[MS_Q_RSZ + 1]
            + commm_ref[MS_Q_RSZ + 2]
        )

        s2 = []
        for d, p in zip((1, 2, 3), z_partners):
            r = copy_m(MS_P_ARZ + 3 - d, p, keep_p)
            r.start()
            s2.append(r)
        for d, p in zip((1, 2, 3), y_partners):
            r = copy_m(MS_Q_ARY + 3 - d, p, keep_q)
            r.start()
            s2.append(r)
        for r in s2:
            r.wait_recv()
        acc_ref[pl.ds(keep_p, 16), :] = (
            acc_ref[pl.ds(keep_p, 16), :]
            + commm_ref[MS_P_ARZ + 0]
            + commm_ref[MS_P_ARZ + 1]
            + commm_ref[MS_P_ARZ + 2]
        )
        acc_ref[pl.ds(keep_q, 16), :] = (
            acc_ref[pl.ds(keep_q, 16), :]
            + commm_ref[MS_Q_ARY + 0]
            + commm_ref[MS_Q_ARY + 1]
            + commm_ref[MS_Q_ARY + 2]
        )

        s3p, s3q = [], []
        for d, p in zip((1, 2, 3), y_partners):
            r = copy_m(MS_P_AGY + 3 - d, p, keep_p)
            r.start()
            s3p.append(r)
        for d, p in zip((1, 2, 3), z_partners):
            r = copy_m(MS_Q_AGZ + 3 - d, p, keep_q)
            r.start()
            s3q.append(r)
        agxp = copy_x(SLOT_AGXP, x_partner, keep_x)
        agxq = copy_x(SLOT_AGXQ, x_partner, keep_x + 64)
        for r in s3p:
            r.wait_recv()
        for d in (1, 2, 3):
            ys = lax.rem(yy + d, 4)
            acc_ref[pl.ds(keep_x + ys * 16, 16), :] = commm_ref[MS_P_AGY + d - 1]
        agxp.start()
        for r in s3q:
            r.wait_recv()
        for d in (1, 2, 3):
            zs = lax.rem(zz + d, 4)
            acc_ref[pl.ds(keep_x + 64 + zs * 16, 16), :] = commm_ref[MS_Q_AGZ + d - 1]
        agxq.start()

        @pl.when(xx == 0)
        def _():
            out_ref[0, :, :] = acc_ref[0:128, :]

        @pl.when(xx == 1)
        def _():
            out_ref[1, :, :] = acc_ref[128:256, :]

        agxp.wait_recv()
        agxq.wait_recv()

        @pl.when(xx == 0)
        def _():
            out_ref[1, 0:64, :] = commx_ref[SLOT_AGXP]
            out_ref[1, 64:128, :] = commx_ref[SLOT_AGXQ]

        @pl.when(xx == 1)
        def _():
            out_ref[0, 0:64, :] = commx_ref[SLOT_AGXP]
            out_ref[0, 64:128, :] = commx_ref[SLOT_AGXQ]

        for r in [rsxp, rsxq] + s1 + s2 + s3p + s3q + [agxp, agxq]:
            r.wait_send()

    return pl.pallas_call(
        body,
        out_shape=jax.ShapeDtypeStruct((B, SQ, D_MODEL), jnp.float32),
        in_specs=[pl.BlockSpec(memory_space=pltpu.VMEM)] * 5,
        out_specs=pl.BlockSpec(memory_space=pltpu.VMEM),
        scratch_shapes=[
            pltpu.VMEM((ROWS, D_MODEL), jnp.float32),
            pltpu.VMEM((4, 64, D_MODEL), jnp.float32),
            pltpu.VMEM((N_MID, 16, D_MODEL), jnp.float32),
            pltpu.SemaphoreType.DMA((N_SEMS,)),
            pltpu.SemaphoreType.DMA((N_SEMS,)),
        ],
        compiler_params=pltpu.CompilerParams(collective_id=0),
    )(x, wq_s, k_l, v_l, wo_s)


# device time: 33892 ns/iter; 1.0466x vs baseline; 1.0007x over previous
import jax
import jax.numpy as jnp
from jax import lax
from jax.experimental import pallas as pl
from jax.experimental.pallas import tpu as pltpu

N_DEV = 32
B = 2
SQ = 128
SKV = 128
HQ_LOCAL = 4
DH = 64
D_MODEL = 512
HD_LOCAL = HQ_LOCAL * DH
NCH = 32

SLOT_RSXP = 0
SLOT_RSXQ = 1
SLOT_AGXP = 2
SLOT_AGXQ = 3
MS_P_RSY = 0
MS_Q_RSZ = 3
MS_P_ARZ = 6
MS_Q_ARY = 9
MS_P_AGY = 12
MS_Q_AGZ = 15
N_MID = 18
N_SEMS = 4 + N_MID


def kernel(x, Wq, K_ext, V_ext, Wo):
    my = lax.axis_index("i")
    wq_s = lax.dynamic_slice_in_dim(Wq, my * HD_LOCAL, HD_LOCAL, axis=1)
    wo_s = lax.dynamic_slice_in_dim(Wo, my * HD_LOCAL, HD_LOCAL, axis=0)
    k_l = K_ext.transpose(0, 2, 1, 3).reshape(B * HQ_LOCAL, SKV, DH)
    v_l = V_ext.transpose(0, 2, 1, 3).reshape(B * HQ_LOCAL, SKV, DH)

    def body(x_ref, wq_ref, k_ref, v_ref, wo_ref, out_ref,
             acc_ref, commx_ref, commm_ref, send_sems, recv_sems):
        my_pos = lax.axis_index("i")
        zz = my_pos // 8
        jj = lax.rem(my_pos, 8)
        yy = jj // 2
        xx = lax.rem(jj + yy, 2)

        def lidx(px, py, pz):
            return pz * 8 + py * 2 + lax.rem(px + py, 2)

        x_partner = lidx(1 - xx, yy, zz)
        y_partners = [lidx(xx, lax.rem(yy + d, 4), zz) for d in (1, 2, 3)]
        z_partners = [lidx(xx, yy, lax.rem(zz + d, 4)) for d in (1, 2, 3)]
        all_partners = [x_partner] + y_partners + z_partners

        barrier = pltpu.get_barrier_semaphore()
        for p in all_partners:
            pl.semaphore_signal(
                barrier, inc=1,
                device_id=(p,), device_id_type=pl.DeviceIdType.MESH,
            )
        pl.semaphore_wait(barrier, len(all_partners))

        def copy_x(slot, dst_dev, src_chunk):
            return pltpu.make_async_remote_copy(
                src_ref=acc_ref.at[pl.ds(src_chunk, 8)],
                dst_ref=commx_ref.at[slot],
                send_sem=send_sems.at[slot],
                recv_sem=recv_sems.at[slot],
                device_id=(dst_dev,),
                device_id_type=pl.DeviceIdType.MESH,
            )

        def copy_m(mslot, dst_dev, src_chunk):
            return pltpu.make_async_remote_copy(
                src_ref=acc_ref.at[pl.ds(src_chunk, 2)],
                dst_ref=commm_ref.at[mslot],
                send_sem=send_sems.at[4 + mslot],
                recv_sem=recv_sems.at[4 + mslot],
                device_id=(dst_dev,),
                device_id_type=pl.DeviceIdType.MESH,
            )

        xs = x_ref[...].reshape(B * SQ, D_MODEL)
        q = jnp.dot(xs, wq_ref[...], preferred_element_type=jnp.float32)

        rowb = lax.broadcasted_iota(jnp.int32, (SQ, SKV), 0) // 64
        colb = lax.broadcasted_iota(jnp.int32, (SQ, SKV), 1) // 64
        mask = rowb == colb

        rsxp = copy_x(SLOT_RSXP, x_partner, (1 - xx) * 16)
        rsxq = copy_x(SLOT_RSXQ, x_partner, (1 - xx) * 16 + 8)

        def partial_b(b):
            ctx_h = []
            for h in range(HQ_LOCAL):
                qbh = q[b * SQ:(b + 1) * SQ, h * DH:(h + 1) * DH]
                kbh = k_ref[b * HQ_LOCAL + h]
                vbh = v_ref[b * HQ_LOCAL + h]
                s = lax.dot_general(
                    qbh, kbh, (((1,), (1,)), ((), ())),
                    preferred_element_type=jnp.float32,
                ) * 0.125
                s = jnp.where(mask, s, -1e9)
                m = jnp.max(s, axis=1, keepdims=True)
                w = jnp.exp(s - m)
                w = w / jnp.sum(w, axis=1, keepdims=True)
                ctx_h.append(jnp.dot(w, vbh, preferred_element_type=jnp.float32))
            ctx = jnp.concatenate(ctx_h, axis=1)
            return jnp.dot(ctx, wo_ref[...],
                           preferred_element_type=jnp.float32)

        acc_ref[0:16] = partial_b(0).reshape(16, 8, D_MODEL)

        @pl.when(xx == 1)
        def _():
            rsxp.start()
            rsxq.start()

        acc_ref[16:32] = partial_b(1).reshape(16, 8, D_MODEL)

        @pl.when(xx == 0)
        def _():
            rsxp.start()
            rsxq.start()

        keep_x = xx * 16
        keep_p = keep_x + yy * 2
        keep_q = keep_x + 8 + zz * 2

        rsxp.wait_recv()
        acc_ref[pl.ds(keep_x, 8)] = (
            acc_ref[pl.ds(keep_x, 8)] + commx_ref[SLOT_RSXP]
        )

        s1 = []
        for d, p in zip((1, 2, 3), y_partners):
            yp = lax.rem(yy + d, 4)
            r = copy_m(MS_P_RSY + 3 - d, p, keep_x + yp * 2)
            r.start()
            s1.append(r)
        rsxq.wait_recv()
        acc_ref[pl.ds(keep_x + 8, 8)] = (
            acc_ref[pl.ds(keep_x + 8, 8)] + commx_ref[SLOT_RSXQ]
        )
        for d, p in zip((1, 2, 3), z_partners):
            zp = lax.rem(zz + d, 4)
            r = copy_m(MS_Q_RSZ + 3 - d, p, keep_x + 8 + zp * 2)
            r.start()
            s1.append(r)
        for r in s1:
            r.wait_recv()
        acc_ref[pl.ds(keep_p, 2)] = (
            acc_ref[pl.ds(keep_p, 2)]
            + commm_ref[MS_P_RSY + 0]
            + commm_ref[MS_P_RSY + 1]
            + commm_ref[MS_P_RSY + 2]
        )
        acc_ref[pl.ds(keep_q, 2)] = (
            acc_ref[pl.ds(keep_q, 2)]
            + commm_ref[MS_Q_RSZ + 0]
            + commm_ref[MS_Q_RSZ + 1]
            + commm_ref[MS_Q_RSZ + 2]
        )

        s2 = []
        for d, p in zip((1, 2, 3), z_partners):
            r = copy_m(MS_P_ARZ + 3 - d, p, keep_p)
            r.start()
            s2.append(r)
        for d, p in zip((1, 2, 3), y_partners):
            r = copy_m(MS_Q_ARY + 3 - d, p, keep_q)
            r.start()
            s2.append(r)
        for r in s2:
            r.wait_recv()
        acc_ref[pl.ds(keep_p, 2)] = (
            acc_ref[pl.ds(keep_p, 2)]
            + commm_ref[MS_P_ARZ + 0]
            + commm_ref[MS_P_ARZ + 1]
            + commm_ref[MS_P_ARZ + 2]
        )
        acc_ref[pl.ds(keep_q, 2)] = (
            acc_ref[pl.ds(keep_q, 2)]
            + commm_ref[MS_Q_ARY + 0]
            + commm_ref[MS_Q_ARY + 1]
            + commm_ref[MS_Q_ARY + 2]
        )

        s3p, s3q = [], []
        for d, p in zip((1, 2, 3), y_partners):
            r = copy_m(MS_P_AGY + 3 - d, p, keep_p)
            r.start()
            s3p.append(r)
        for d, p in zip((1, 2, 3), z_partners):
            r = copy_m(MS_Q_AGZ + 3 - d, p, keep_q)
            r.start()
            s3q.append(r)
        agxp = copy_x(SLOT_AGXP, x_partner, keep_x)
        agxq = copy_x(SLOT_AGXQ, x_partner, keep_x + 8)
        for r in s3p:
            r.wait_recv()
        for d in (1, 2, 3):
            ys = lax.rem(yy + d, 4)
            acc_ref[pl.ds(keep_x + ys * 2, 2)] = commm_ref[MS_P_AGY + d - 1]
        agxp.start()
        for r in s3q:
            r.wait_recv()
        for d in (1, 2, 3):
            zs = lax.rem(zz + d, 4)
            acc_ref[pl.ds(keep_x + 8 + zs * 2, 2)] = commm_ref[MS_Q_AGZ + d - 1]
        agxq.start()

        @pl.when(xx == 0)
        def _():
            out_ref[0, :, :] = acc_ref[0:16].reshape(SQ, D_MODEL)

        @pl.when(xx == 1)
        def _():
            out_ref[1, :, :] = acc_ref[16:32].reshape(SQ, D_MODEL)

        agxp.wait_recv()
        agxq.wait_recv()

        @pl.when(xx == 0)
        def _():
            out_ref[1, 0:64, :] = commx_ref[SLOT_AGXP].reshape(64, D_MODEL)
            out_ref[1, 64:128, :] = commx_ref[SLOT_AGXQ].reshape(64, D_MODEL)

        @pl.when(xx == 1)
        def _():
            out_ref[0, 0:64, :] = commx_ref[SLOT_AGXP].reshape(64, D_MODEL)
            out_ref[0, 64:128, :] = commx_ref[SLOT_AGXQ].reshape(64, D_MODEL)

        for r in [rsxp, rsxq] + s1 + s2 + s3p + s3q + [agxp, agxq]:
            r.wait_send()

    return pl.pallas_call(
        body,
        out_shape=jax.ShapeDtypeStruct((B, SQ, D_MODEL), jnp.float32),
        in_specs=[pl.BlockSpec(memory_space=pltpu.VMEM)] * 5,
        out_specs=pl.BlockSpec(memory_space=pltpu.VMEM),
        scratch_shapes=[
            pltpu.VMEM((NCH, 8, D_MODEL), jnp.float32),
            pltpu.VMEM((4, 8, 8, D_MODEL), jnp.float32),
            pltpu.VMEM((N_MID, 2, 8, D_MODEL), jnp.float32),
            pltpu.SemaphoreType.DMA((N_SEMS,)),
            pltpu.SemaphoreType.DMA((N_SEMS,)),
        ],
        compiler_params=pltpu.CompilerParams(collective_id=0),
    )(x, wq_s, k_l, v_l, wo_s)


# device time: 33765 ns/iter; 1.0506x vs baseline; 1.0038x over previous
import jax
import jax.numpy as jnp
from jax import lax
from jax.experimental import pallas as pl
from jax.experimental.pallas import tpu as pltpu

N_DEV = 32
B = 2
SQ = 128
SKV = 128
HQ_LOCAL = 4
DH = 64
D_MODEL = 512
HD_LOCAL = HQ_LOCAL * DH
NCH = 32

SLOT_RSXP = 0
SLOT_RSXQ = 1
SLOT_AGXP = 2
SLOT_AGXQ = 3
MS_P_RSY = 0
MS_Q_RSZ = 3
MS_P_ARZ = 6
MS_Q_ARY = 9
MS_P_AGY = 12
MS_Q_AGZ = 15
N_MID = 18
N_SEMS = 4 + N_MID


def kernel(x, Wq, K_ext, V_ext, Wo):
    my = lax.axis_index("i")
    wq_s = lax.dynamic_slice_in_dim(Wq, my * HD_LOCAL, HD_LOCAL, axis=1)
    wo_s = lax.dynamic_slice_in_dim(Wo, my * HD_LOCAL, HD_LOCAL, axis=0)
    k_l = K_ext.transpose(0, 2, 1, 3).reshape(B * HQ_LOCAL, SKV, DH)
    v_l = V_ext.transpose(0, 2, 1, 3).reshape(B * HQ_LOCAL, SKV, DH)

    def body(x_ref, wq_ref, k_ref, v_ref, wo_ref, out_ref,
             acc_ref, commx_ref, commm_ref, send_sems, recv_sems):
        my_pos = lax.axis_index("i")
        zz = my_pos // 8
        jj = lax.rem(my_pos, 8)
        yy = jj // 2
        xx = lax.rem(jj + yy, 2)

        def lidx(px, py, pz):
            return pz * 8 + py * 2 + lax.rem(px + py, 2)

        x_partner = lidx(1 - xx, yy, zz)
        y_partners = [lidx(xx, lax.rem(yy + d, 4), zz) for d in (1, 2, 3)]
        z_partners = [lidx(xx, yy, lax.rem(zz + d, 4)) for d in (1, 2, 3)]
        all_partners = [x_partner] + y_partners + z_partners

        barrier = pltpu.get_barrier_semaphore()
        for p in all_partners:
            pl.semaphore_signal(
                barrier, inc=1,
                device_id=(p,), device_id_type=pl.DeviceIdType.MESH,
            )
        pl.semaphore_wait(barrier, len(all_partners))

        def copy_x(slot, dst_dev, src_chunk):
            return pltpu.make_async_remote_copy(
                src_ref=acc_ref.at[pl.ds(src_chunk, 8)],
                dst_ref=commx_ref.at[slot],
                send_sem=send_sems.at[slot],
                recv_sem=recv_sems.at[slot],
                device_id=(dst_dev,),
                device_id_type=pl.DeviceIdType.MESH,
            )

        def copy_m(mslot, dst_dev, src_chunk):
            return pltpu.make_async_remote_copy(
                src_ref=acc_ref.at[pl.ds(src_chunk, 2)],
                dst_ref=commm_ref.at[mslot],
                send_sem=send_sems.at[4 + mslot],
                recv_sem=recv_sems.at[4 + mslot],
                device_id=(dst_dev,),
                device_id_type=pl.DeviceIdType.MESH,
            )

        xs = x_ref[...].reshape(B * SQ, D_MODEL)
        q = jnp.dot(xs, wq_ref[...], preferred_element_type=jnp.float32)

        rowb = lax.broadcasted_iota(jnp.int32, (SQ, SKV), 0) // 64
        colb = lax.broadcasted_iota(jnp.int32, (SQ, SKV), 1) // 64
        mask = rowb == colb

        rsxp = copy_x(SLOT_RSXP, x_partner, (1 - xx) * 16)
        rsxq = copy_x(SLOT_RSXQ, x_partner, (1 - xx) * 16 + 8)

        def partial_b(b):
            ctx_h = []
            for h in range(HQ_LOCAL):
                qbh = q[b * SQ:(b + 1) * SQ, h * DH:(h + 1) * DH]
                kbh = k_ref[b * HQ_LOCAL + h]
                vbh = v_ref[b * HQ_LOCAL + h]
                s = lax.dot_general(
                    qbh, kbh, (((1,), (1,)), ((), ())),
                    preferred_element_type=jnp.float32,
                ) * 0.125
                s = jnp.where(mask, s, -1e9)
                m = jnp.max(s, axis=1, keepdims=True)
                w = jnp.exp(s - m)
                w = w / jnp.sum(w, axis=1, keepdims=True)
                ctx_h.append(jnp.dot(w, vbh, preferred_element_type=jnp.float32))
            ctx = jnp.concatenate(ctx_h, axis=1)
            return jnp.dot(ctx, wo_ref[...],
                           preferred_element_type=jnp.float32)

        acc_ref[0:16] = partial_b(0).reshape(16, 8, D_MODEL)
        acc_ref[16:32] = partial_b(1).reshape(16, 8, D_MODEL)

        rsxp.start()
        rsxq.start()

        keep_x = xx * 16
        keep_p = keep_x + yy * 2
        keep_q = keep_x + 8 + zz * 2

        rsxp.wait_recv()
        acc_ref[pl.ds(keep_x, 8)] = (
            acc_ref[pl.ds(keep_x, 8)] + commx_ref[SLOT_RSXP]
        )

        s1 = []
        for d, p in zip((1, 2, 3), y_partners):
            yp = lax.rem(yy + d, 4)
            r = copy_m(MS_P_RSY + 3 - d, p, keep_x + yp * 2)
            r.start()
            s1.append(r)
        rsxq.wait_recv()
        acc_ref[pl.ds(keep_x + 8, 8)] = (
            acc_ref[pl.ds(keep_x + 8, 8)] + commx_ref[SLOT_RSXQ]
        )
        for d, p in zip((1, 2, 3), z_partners):
            zp = lax.rem(zz + d, 4)
            r = copy_m(MS_Q_RSZ + 3 - d, p, keep_x + 8 + zp * 2)
            r.start()
            s1.append(r)
        for r in s1:
            r.wait_recv()
        acc_ref[pl.ds(keep_p, 2)] = (
            acc_ref[pl.ds(keep_p, 2)]
            + commm_ref[MS_P_RSY + 0]
            + commm_ref[MS_P_RSY + 1]
            + commm_ref[MS_P_RSY + 2]
        )
        acc_ref[pl.ds(keep_q, 2)] = (
            acc_ref[pl.ds(keep_q, 2)]
            + commm_ref[MS_Q_RSZ + 0]
            + commm_ref[MS_Q_RSZ + 1]
            + commm_ref[MS_Q_RSZ + 2]
        )

        s2 = []
        for d, p in zip((1, 2, 3), z_partners):
            r = copy_m(MS_P_ARZ + 3 - d, p, keep_p)
            r.start()
            s2.append(r)
        for d, p in zip((1, 2, 3), y_partners):
            r = copy_m(MS_Q_ARY + 3 - d, p, keep_q)
            r.start()
            s2.append(r)
        for r in s2:
            r.wait_recv()
        acc_ref[pl.ds(keep_p, 2)] = (
            acc_ref[pl.ds(keep_p, 2)]
            + commm_ref[MS_P_ARZ + 0]
            + commm_ref[MS_P_ARZ + 1]
            + commm_ref[MS_P_ARZ + 2]
        )
        acc_ref[pl.ds(keep_q, 2)] = (
            acc_ref[pl.ds(keep_q, 2)]
            + commm_ref[MS_Q_ARY + 0]
            + commm_ref[MS_Q_ARY + 1]
            + commm_ref[MS_Q_ARY + 2]
        )

        s3p, s3q = [], []
        for d, p in zip((1, 2, 3), y_partners):
            r = copy_m(MS_P_AGY + 3 - d, p, keep_p)
            r.start()
            s3p.append(r)
        for d, p in zip((1, 2, 3), z_partners):
            r = copy_m(MS_Q_AGZ + 3 - d, p, keep_q)
            r.start()
            s3q.append(r)
        agxp = copy_x(SLOT_AGXP, x_partner, keep_x)
        agxq = copy_x(SLOT_AGXQ, x_partner, keep_x + 8)
        for r in s3p:
            r.wait_recv()
        for d in (1, 2, 3):
            ys = lax.rem(yy + d, 4)
            acc_ref[pl.ds(keep_x + ys * 2, 2)] = commm_ref[MS_P_AGY + d - 1]
        agxp.start()
        for r in s3q:
            r.wait_recv()
        for d in (1, 2, 3):
            zs = lax.rem(zz + d, 4)
            acc_ref[pl.ds(keep_x + 8 + zs * 2, 2)] = commm_ref[MS_Q_AGZ + d - 1]
        agxq.start()

        agxp.wait_recv()
        agxq.wait_recv()
        acc_ref[pl.ds((1 - xx) * 16, 8)] = commx_ref[SLOT_AGXP]
        acc_ref[pl.ds((1 - xx) * 16 + 8, 8)] = commx_ref[SLOT_AGXQ]
        out_ref[0, :, :] = acc_ref[0:16].reshape(SQ, D_MODEL)
        out_ref[1, :, :] = acc_ref[16:32].reshape(SQ, D_MODEL)

        for r in [rsxp, rsxq] + s1 + s2 + s3p + s3q + [agxp, agxq]:
            r.wait_send()

    return pl.pallas_call(
        body,
        out_shape=jax.ShapeDtypeStruct((B, SQ, D_MODEL), jnp.float32),
        in_specs=[pl.BlockSpec(memory_space=pltpu.VMEM)] * 5,
        out_specs=pl.BlockSpec(memory_space=pltpu.VMEM),
        scratch_shapes=[
            pltpu.VMEM((NCH, 8, D_MODEL), jnp.float32),
            pltpu.VMEM((4, 8, 8, D_MODEL), jnp.float32),
            pltpu.VMEM((N_MID, 2, 8, D_MODEL), jnp.float32),
            pltpu.SemaphoreType.DMA((N_SEMS,)),
            pltpu.SemaphoreType.DMA((N_SEMS,)),
        ],
        compiler_params=pltpu.CompilerParams(collective_id=0),
    )(x, wq_s, k_l, v_l, wo_s)
